# Initial kernel scaffold; baseline (speedup 1.0000x reference)
import jax
import jax.numpy as jnp
from jax.experimental import pallas as pl


def kernel(x, edge_index1, edge_index2, W1, W2):
    raise NotImplementedError("write your pallas kernel here")


# speedup vs baseline: 4.8482x; 4.8482x over previous
import functools

import jax
import jax.numpy as jnp
from jax import lax
from jax.experimental import pallas as pl
from jax.experimental.pallas import tpu as pltpu
from jax.experimental.pallas import tpu_sc as plsc

_N = 10000
_E = 160000
_D = 128

_NSUB = 16
_CHUNK = 80
_EDGES_PER_SUB = _E // _NSUB
_CHUNKS = _EDGES_PER_SUB // _CHUNK
_ROWS_PER_SUB = 624
_ZROWS = 160


def _sc_segment_sums(x, ei):
    mesh = plsc.VectorSubcoreMesh(core_axis_name="c", subcore_axis_name="s")

    @functools.partial(
        pl.kernel,
        mesh=mesh,
        out_type=jax.ShapeDtypeStruct((2, _N, _D), jnp.float32),
        scratch_types=[
            pltpu.VMEM((_CHUNK,), jnp.int32),
            pltpu.VMEM((_CHUNK,), jnp.int32),
            pltpu.VMEM((_CHUNK, _D), jnp.float32),
            pltpu.VMEM((_ZROWS, _D), jnp.float32),
            pltpu.VMEM_SHARED((_N, _D), jnp.float32),
            pltpu.SemaphoreType.DMA,
        ],
    )
    def k(x_hbm, ei_hbm, out_hbm, src_v, dst_v, rows_v, zero_v, acc_sh, sem):
        c = lax.axis_index("c")
        s = lax.axis_index("s")

        zvec = jnp.zeros((16,), jnp.float32)

        def zrow(i, carry):
            for k8 in range(_D // 16):
                zero_v[i, pl.ds(k8 * 16, 16)] = zvec
            return carry

        lax.fori_loop(0, _ZROWS, zrow, None)

        row0 = s * _ROWS_PER_SUB
        for j in range(4):
            pltpu.sync_copy(zero_v, acc_sh.at[pl.ds(row0 + j * _ZROWS, _ZROWS)])
        plsc.subcore_barrier()

        ebase = s * _EDGES_PER_SUB

        def body(g, carry):
            off = ebase + g * _CHUNK
            pltpu.sync_copy(ei_hbm.at[pl.ds(2 * c * _E + off, _CHUNK)], src_v)
            pltpu.sync_copy(ei_hbm.at[pl.ds((2 * c + 1) * _E + off, _CHUNK)], dst_v)
            pltpu.async_copy(x_hbm.at[src_v], rows_v, sem).wait()
            pltpu.sync_copy(rows_v, acc_sh.at[dst_v], add=True)
            return carry

        lax.fori_loop(0, _CHUNKS, body, None)
        plsc.subcore_barrier()

        pltpu.sync_copy(
            acc_sh.at[pl.ds(row0, _ROWS_PER_SUB)],
            out_hbm.at[c, pl.ds(row0, _ROWS_PER_SUB)],
        )
        @pl.when(s == 0)
        def _():
            pltpu.sync_copy(
                acc_sh.at[pl.ds(_NSUB * _ROWS_PER_SUB, _N - _NSUB * _ROWS_PER_SUB)],
                out_hbm.at[c, pl.ds(_NSUB * _ROWS_PER_SUB, _N - _NSUB * _ROWS_PER_SUB)],
            )

    return k(x, ei)


def _tc_finalize(parts, W):
    bm = 1000

    def body(p_ref, w_ref, o_ref):
        r = jnp.dot(p_ref[0], w_ref[0], preferred_element_type=jnp.float32)
        r = r + jnp.dot(p_ref[1], w_ref[1], preferred_element_type=jnp.float32)
        norm = jnp.sqrt(jnp.sum(r * r, axis=1, keepdims=True))
        r = r / jnp.maximum(norm, 1e-12)
        o_ref[...] = jnp.maximum(r, 0.0)

    return pl.pallas_call(
        body,
        grid=(_N // bm,),
        in_specs=[
            pl.BlockSpec((2, bm, _D), lambda i: (0, i, 0)),
            pl.BlockSpec((2, _D, _D), lambda i: (0, 0, 0)),
        ],
        out_specs=pl.BlockSpec((bm, _D), lambda i: (i, 0)),
        out_shape=jax.ShapeDtypeStruct((_N, _D), jnp.float32),
    )(parts, W)


def kernel(x, edge_index1, edge_index2, W1, W2):
    ei = jnp.concatenate(
        [edge_index1[0], edge_index1[1], edge_index2[0], edge_index2[1]]
    )
    W = jnp.stack([W1, W2])
    parts = _sc_segment_sums(x, ei)
    return _tc_finalize(parts, W)


# speedup vs baseline: 12.7631x; 2.6326x over previous
import functools

import jax
import jax.numpy as jnp
from jax import lax
from jax.experimental import pallas as pl
from jax.experimental.pallas import tpu as pltpu
from jax.experimental.pallas import tpu_sc as plsc

_N = 10000
_E = 160000
_D = 128

_NSUB = 16
_CHUNK = 80
_EDGES_PER_SUB = _E // _NSUB
_CHUNKS = _EDGES_PER_SUB // _CHUNK
_ROWS_PER_SUB = 624
_ZROWS = 40
_NBUF = 3


def _sc_segment_sums(x, ei):
    mesh = plsc.VectorSubcoreMesh(core_axis_name="c", subcore_axis_name="s")

    @functools.partial(
        pl.kernel,
        mesh=mesh,
        out_type=jax.ShapeDtypeStruct((2, _N, _D), jnp.float32),
        scratch_types=[
            pltpu.VMEM((_EDGES_PER_SUB,), jnp.int32),
            [pltpu.VMEM((_CHUNK,), jnp.int32) for _ in range(_NBUF)],
            [pltpu.VMEM((_CHUNK, _D), jnp.float32) for _ in range(_NBUF)],
            pltpu.VMEM((_ZROWS, _D), jnp.float32),
            pltpu.VMEM_SHARED((_N, _D), jnp.float32),
            [pltpu.SemaphoreType.DMA for _ in range(_NBUF)],
            [pltpu.SemaphoreType.DMA for _ in range(_NBUF)],
        ],
    )
    def k(x_hbm, ei_hbm, out_hbm, src_v, dst_bufs, row_bufs, zero_v, acc_sh,
          isems, gsems):
        c = lax.axis_index("c")
        s = lax.axis_index("s")

        zvec = jnp.zeros((16,), jnp.float32)

        def zrow(i, carry):
            for k8 in range(_D // 16):
                zero_v[i, pl.ds(k8 * 16, 16)] = zvec
            return carry

        lax.fori_loop(0, _ZROWS, zrow, None)

        row0 = s * _ROWS_PER_SUB
        for j in range(640 // _ZROWS):
            pltpu.sync_copy(zero_v, acc_sh.at[pl.ds(row0 + j * _ZROWS, _ZROWS)])
        plsc.subcore_barrier()

        ebase = s * _EDGES_PER_SUB

        pltpu.sync_copy(ei_hbm.at[pl.ds(2 * c * _E + ebase, _EDGES_PER_SUB)],
                        src_v)
        dbase = (2 * c + 1) * _E + ebase

        def issue(b, g):
            pltpu.make_async_copy(
                ei_hbm.at[pl.ds(dbase + g * _CHUNK, _CHUNK)],
                dst_bufs[b], isems[b]).start()
            pltpu.make_async_copy(
                x_hbm.at[src_v.at[pl.ds(g * _CHUNK, _CHUNK)]],
                row_bufs[b], gsems[b]).start()

        def drain_and_scatter(b):
            pltpu.make_async_copy(
                ei_hbm.at[pl.ds(dbase, _CHUNK)], dst_bufs[b], isems[b]).wait()
            pltpu.make_async_copy(
                x_hbm.at[src_v.at[pl.ds(0, _CHUNK)]], row_bufs[b],
                gsems[b]).wait()
            pltpu.sync_copy(row_bufs[b], acc_sh.at[dst_bufs[b]], add=True)

        for b in range(_NBUF):
            issue(b, b)

        n_full = _CHUNKS // _NBUF

        def body(i, carry):
            for b in range(_NBUF):
                g = _NBUF * i + b
                drain_and_scatter(b)

                @pl.when(g + _NBUF < _CHUNKS)
                def _():
                    issue(b, g + _NBUF)
            return carry

        lax.fori_loop(0, n_full, body, None)
        for b in range(_CHUNKS - n_full * _NBUF):
            drain_and_scatter(b)
        plsc.subcore_barrier()

        pltpu.sync_copy(
            acc_sh.at[pl.ds(row0, _ROWS_PER_SUB)],
            out_hbm.at[c, pl.ds(row0, _ROWS_PER_SUB)],
        )
        @pl.when(s == 0)
        def _():
            pltpu.sync_copy(
                acc_sh.at[pl.ds(_NSUB * _ROWS_PER_SUB, _N - _NSUB * _ROWS_PER_SUB)],
                out_hbm.at[c, pl.ds(_NSUB * _ROWS_PER_SUB, _N - _NSUB * _ROWS_PER_SUB)],
            )

    return k(x, ei)


def _tc_finalize(parts, W):
    bm = 1000

    def body(p_ref, w_ref, o_ref):
        r = jnp.dot(p_ref[0], w_ref[0], preferred_element_type=jnp.float32)
        r = r + jnp.dot(p_ref[1], w_ref[1], preferred_element_type=jnp.float32)
        norm = jnp.sqrt(jnp.sum(r * r, axis=1, keepdims=True))
        r = r / jnp.maximum(norm, 1e-12)
        o_ref[...] = jnp.maximum(r, 0.0)

    return pl.pallas_call(
        body,
        grid=(_N // bm,),
        in_specs=[
            pl.BlockSpec((2, bm, _D), lambda i: (0, i, 0)),
            pl.BlockSpec((2, _D, _D), lambda i: (0, 0, 0)),
        ],
        out_specs=pl.BlockSpec((bm, _D), lambda i: (i, 0)),
        out_shape=jax.ShapeDtypeStruct((_N, _D), jnp.float32),
    )(parts, W)


def kernel(x, edge_index1, edge_index2, W1, W2):
    ei = jnp.concatenate(
        [edge_index1[0], edge_index1[1], edge_index2[0], edge_index2[1]]
    )
    W = jnp.stack([W1, W2])
    parts = _sc_segment_sums(x, ei)
    return _tc_finalize(parts, W)


# speedup vs baseline: 13.5635x; 1.0627x over previous
import functools

import jax
import jax.numpy as jnp
from jax import lax
from jax.experimental import pallas as pl
from jax.experimental.pallas import tpu as pltpu
from jax.experimental.pallas import tpu_sc as plsc

_N = 10000
_E = 160000
_D = 128

_NSUB = 16
_CHUNK = 80
_EDGES_PER_SUB = _E // _NSUB
_CHUNKS = _EDGES_PER_SUB // _CHUNK
_ROWS_PER_SUB = 624
_ZROWS = 40
_NBUF = 3


def _sc_segment_sums(x, ei1, ei2):
    mesh = plsc.VectorSubcoreMesh(core_axis_name="c", subcore_axis_name="s")

    @functools.partial(
        pl.kernel,
        mesh=mesh,
        out_type=jax.ShapeDtypeStruct((2, _N, _D), jnp.float32),
        scratch_types=[
            pltpu.VMEM((_EDGES_PER_SUB,), jnp.int32),
            [pltpu.VMEM((_CHUNK,), jnp.int32) for _ in range(_NBUF)],
            [pltpu.VMEM((_CHUNK, _D), jnp.float32) for _ in range(_NBUF)],
            pltpu.VMEM((_ZROWS, _D), jnp.float32),
            pltpu.VMEM_SHARED((_N, _D), jnp.float32),
            [pltpu.SemaphoreType.DMA for _ in range(_NBUF)],
            [pltpu.SemaphoreType.DMA for _ in range(_NBUF)],
            pltpu.SemaphoreType.DMA,
            pltpu.SemaphoreType.DMA,
        ],
    )
    def k(x_hbm, ei1_hbm, ei2_hbm, out_hbm, src_v, dst_bufs, row_bufs, zero_v,
          acc_sh, isems, gsems, ssem, zsem):
        c = lax.axis_index("c")
        s = lax.axis_index("s")

        ebase = s * _EDGES_PER_SUB
        dbase = _E + ebase

        @pl.when(c == 0)
        def _():
            pltpu.make_async_copy(
                ei1_hbm.at[pl.ds(ebase, _EDGES_PER_SUB)], src_v, ssem).start()

        @pl.when(c == 1)
        def _():
            pltpu.make_async_copy(
                ei2_hbm.at[pl.ds(ebase, _EDGES_PER_SUB)], src_v, ssem).start()

        zvec = jnp.zeros((16,), jnp.float32)

        def zrow(i, carry):
            for k8 in range(_D // 16):
                zero_v[i, pl.ds(k8 * 16, 16)] = zvec
            return carry

        lax.fori_loop(0, _ZROWS, zrow, None)

        pltpu.make_async_copy(
            ei1_hbm.at[pl.ds(0, _EDGES_PER_SUB)], src_v, ssem).wait()

        def issue(ei_hbm, b, g):
            pltpu.make_async_copy(
                ei_hbm.at[pl.ds(dbase + g * _CHUNK, _CHUNK)],
                dst_bufs[b], isems[b]).start()
            pltpu.make_async_copy(
                x_hbm.at[src_v.at[pl.ds(g * _CHUNK, _CHUNK)]],
                row_bufs[b], gsems[b]).start()

        def drain_and_scatter(b):
            pltpu.make_async_copy(
                ei1_hbm.at[pl.ds(dbase, _CHUNK)], dst_bufs[b], isems[b]).wait()
            pltpu.make_async_copy(
                x_hbm.at[src_v.at[pl.ds(0, _CHUNK)]], row_bufs[b],
                gsems[b]).wait()
            pltpu.sync_copy(row_bufs[b], acc_sh.at[dst_bufs[b]], add=True)

        @pl.when(c == 0)
        def _():
            for b in range(_NBUF):
                issue(ei1_hbm, b, b)

        @pl.when(c == 1)
        def _():
            for b in range(_NBUF):
                issue(ei2_hbm, b, b)

        row0 = s * _ROWS_PER_SUB
        for j in range(640 // _ZROWS):
            pltpu.make_async_copy(
                zero_v, acc_sh.at[pl.ds(row0 + j * _ZROWS, _ZROWS)],
                zsem).start()
        for j in range(640 // _ZROWS):
            pltpu.make_async_copy(
                zero_v, acc_sh.at[pl.ds(row0 + j * _ZROWS, _ZROWS)],
                zsem).wait()
        plsc.subcore_barrier()

        n_full = _CHUNKS // _NBUF

        def main(ei_hbm):
            def body(i, carry):
                for b in range(_NBUF):
                    g = _NBUF * i + b
                    drain_and_scatter(b)

                    @pl.when(g + _NBUF < _CHUNKS)
                    def _():
                        issue(ei_hbm, b, g + _NBUF)
                return carry

            lax.fori_loop(0, n_full, body, None)
            for b in range(_CHUNKS - n_full * _NBUF):
                drain_and_scatter(b)

        @pl.when(c == 0)
        def _():
            main(ei1_hbm)

        @pl.when(c == 1)
        def _():
            main(ei2_hbm)

        plsc.subcore_barrier()

        pltpu.sync_copy(
            acc_sh.at[pl.ds(row0, _ROWS_PER_SUB)],
            out_hbm.at[c, pl.ds(row0, _ROWS_PER_SUB)],
        )
        @pl.when(s == 0)
        def _():
            pltpu.sync_copy(
                acc_sh.at[pl.ds(_NSUB * _ROWS_PER_SUB, _N - _NSUB * _ROWS_PER_SUB)],
                out_hbm.at[c, pl.ds(_NSUB * _ROWS_PER_SUB, _N - _NSUB * _ROWS_PER_SUB)],
            )

    return k(x, ei1, ei2)


def _tc_finalize(parts, W1, W2):
    bm = 1000

    def body(p_ref, w1_ref, w2_ref, o_ref):
        r = jnp.dot(p_ref[0], w1_ref[...], preferred_element_type=jnp.float32)
        r = r + jnp.dot(p_ref[1], w2_ref[...], preferred_element_type=jnp.float32)
        norm = jnp.sqrt(jnp.sum(r * r, axis=1, keepdims=True))
        r = r / jnp.maximum(norm, 1e-12)
        o_ref[...] = jnp.maximum(r, 0.0)

    return pl.pallas_call(
        body,
        grid=(_N // bm,),
        in_specs=[
            pl.BlockSpec((2, bm, _D), lambda i: (0, i, 0)),
            pl.BlockSpec((_D, _D), lambda i: (0, 0)),
            pl.BlockSpec((_D, _D), lambda i: (0, 0)),
        ],
        out_specs=pl.BlockSpec((bm, _D), lambda i: (i, 0)),
        out_shape=jax.ShapeDtypeStruct((_N, _D), jnp.float32),
    )(parts, W1, W2)


def kernel(x, edge_index1, edge_index2, W1, W2):
    parts = _sc_segment_sums(
        x, edge_index1.reshape(2 * _E), edge_index2.reshape(2 * _E)
    )
    return _tc_finalize(parts, W1, W2)


# speedup vs baseline: 14.1327x; 1.0420x over previous
import functools

import jax
import jax.numpy as jnp
from jax import lax
from jax.experimental import pallas as pl
from jax.experimental.pallas import tpu as pltpu
from jax.experimental.pallas import tpu_sc as plsc

_N = 10000
_E = 160000
_D = 128

_NSUB = 16
_CHUNK = 128
_NCHUNKS = _E // _CHUNK
_BASE_CH = _NCHUNKS // _NSUB
_EXTRA = _NCHUNKS - _BASE_CH * _NSUB
_ROWS_PER_SUB = 624
_NROW = 3
_NIDX = 5
_UNROLL = 15


def _sc_segment_sums(x, ei1, ei2):
    mesh = plsc.VectorSubcoreMesh(core_axis_name="c", subcore_axis_name="s")

    @functools.partial(
        pl.kernel,
        mesh=mesh,
        out_type=jax.ShapeDtypeStruct((2, _N, _D), jnp.float32),
        scratch_types=[
            [pltpu.VMEM((2, _CHUNK), jnp.int32) for _ in range(_NIDX)],
            [pltpu.VMEM((_CHUNK, _D), jnp.float32) for _ in range(_NROW)],
            pltpu.VMEM_SHARED((_N, _D), jnp.float32),
            [pltpu.SemaphoreType.DMA for _ in range(_NIDX)],
            [pltpu.SemaphoreType.DMA for _ in range(_NROW)],
            pltpu.SemaphoreType.DMA,
        ],
    )
    def k(x_hbm, ei1_hbm, ei2_hbm, out_hbm, idx_bufs, row_bufs, acc_sh,
          isems, gsems, zsem):
        c = lax.axis_index("c")
        s = lax.axis_index("s")

        first = s * _BASE_CH + jnp.minimum(s, _EXTRA)
        n_s = _BASE_CH + jnp.where(s < _EXTRA, 1, 0)

        def issue_idx(ei_hbm, g, si):
            off = (first + g) * _CHUNK
            pltpu.make_async_copy(
                ei_hbm.at[pl.ds(0, 2), pl.ds(off, _CHUNK)],
                idx_bufs[si], isems[si]).start()

        def wait_idx(ei_hbm, si):
            pltpu.make_async_copy(
                ei_hbm.at[pl.ds(0, 2), pl.ds(0, _CHUNK)],
                idx_bufs[si], isems[si]).wait()

        def issue_gather(si, ri):
            pltpu.make_async_copy(
                x_hbm.at[idx_bufs[si].at[0]],
                row_bufs[ri], gsems[ri]).start()

        def wait_gather_scatter(si, ri):
            pltpu.make_async_copy(
                x_hbm.at[idx_bufs[si].at[0]],
                row_bufs[ri], gsems[ri]).wait()
            pltpu.sync_copy(row_bufs[ri],
                            acc_sh.at[idx_bufs[si].at[1]], add=True)

        def prologue(ei_hbm):
            for g in range(_NIDX):
                issue_idx(ei_hbm, g, g)

        def after_zero(ei_hbm):
            for g in range(_NROW):
                wait_idx(ei_hbm, g)
                issue_gather(g, g)

        def main(ei_hbm):
            def body(i, carry):
                for u in range(_UNROLL):
                    g = _UNROLL * i + u

                    @pl.when(g < n_s)
                    def _():
                        wait_gather_scatter(u % _NIDX, u % _NROW)

                    @pl.when(g + _NIDX < n_s)
                    def _():
                        issue_idx(ei_hbm, g + _NIDX, u % _NIDX)

                    @pl.when(g + _NROW < n_s)
                    def _():
                        wait_idx(ei_hbm, (u + _NROW) % _NIDX)
                        issue_gather((u + _NROW) % _NIDX, u % _NROW)
                return carry

            lax.fori_loop(0, (_BASE_CH + 1 + _UNROLL - 1) // _UNROLL,
                          body, None)

        @pl.when(c == 0)
        def _():
            prologue(ei1_hbm)

        @pl.when(c == 1)
        def _():
            prologue(ei2_hbm)

        zvec = jnp.zeros((16,), jnp.float32)

        def zrow(i, carry):
            for k8 in range(_D // 16):
                row_bufs[0][i, pl.ds(k8 * 16, 16)] = zvec
            return carry

        lax.fori_loop(0, _CHUNK, zrow, None)

        row0 = s * _ROWS_PER_SUB
        for j in range(5):
            pltpu.make_async_copy(
                row_bufs[0], acc_sh.at[pl.ds(row0 + j * _CHUNK, _CHUNK)],
                zsem).start()
        for j in range(5):
            pltpu.make_async_copy(
                row_bufs[0], acc_sh.at[pl.ds(row0 + j * _CHUNK, _CHUNK)],
                zsem).wait()

        @pl.when(c == 0)
        def _():
            after_zero(ei1_hbm)

        @pl.when(c == 1)
        def _():
            after_zero(ei2_hbm)

        plsc.subcore_barrier()

        @pl.when(c == 0)
        def _():
            main(ei1_hbm)

        @pl.when(c == 1)
        def _():
            main(ei2_hbm)

        plsc.subcore_barrier()

        pltpu.sync_copy(
            acc_sh.at[pl.ds(row0, _ROWS_PER_SUB)],
            out_hbm.at[c, pl.ds(row0, _ROWS_PER_SUB)],
        )
        @pl.when(s == 0)
        def _():
            pltpu.sync_copy(
                acc_sh.at[pl.ds(_NSUB * _ROWS_PER_SUB, _N - _NSUB * _ROWS_PER_SUB)],
                out_hbm.at[c, pl.ds(_NSUB * _ROWS_PER_SUB, _N - _NSUB * _ROWS_PER_SUB)],
            )

    return k(x, ei1, ei2)


def _tc_finalize(parts, W1, W2):
    bm = 2000

    def body(p_ref, w1_ref, w2_ref, o_ref):
        r = jnp.dot(p_ref[0], w1_ref[...], preferred_element_type=jnp.float32)
        r = r + jnp.dot(p_ref[1], w2_ref[...], preferred_element_type=jnp.float32)
        norm = jnp.sqrt(jnp.sum(r * r, axis=1, keepdims=True))
        r = r / jnp.maximum(norm, 1e-12)
        o_ref[...] = jnp.maximum(r, 0.0)

    return pl.pallas_call(
        body,
        grid=(_N // bm,),
        in_specs=[
            pl.BlockSpec((2, bm, _D), lambda i: (0, i, 0)),
            pl.BlockSpec((_D, _D), lambda i: (0, 0)),
            pl.BlockSpec((_D, _D), lambda i: (0, 0)),
        ],
        out_specs=pl.BlockSpec((bm, _D), lambda i: (i, 0)),
        out_shape=jax.ShapeDtypeStruct((_N, _D), jnp.float32),
    )(parts, W1, W2)


def kernel(x, edge_index1, edge_index2, W1, W2):
    parts = _sc_segment_sums(x, edge_index1, edge_index2)
    return _tc_finalize(parts, W1, W2)
